# baseline (speedup 1.0000x reference)
import math

import jax
import jax.numpy as jnp
from jax.experimental import pallas as pl

_CK = 64
_HW = 1024
_T = 16384
_TOPK = 30
_QT = 128
_NQT = _HW // _QT
_KC = 1024
_NKC = _T // _KC
_CV2 = 1024


def _affinity_kernel(qk_ref, qs_ref, mk_ref, shr_ref, aff_ref):
    qk = qk_ref[...]
    qs = qs_ref[...]
    mk = mk_ref[...]
    a_sq = jax.lax.dot_general(
        qs, mk * mk, (((0,), (0,)), ((), ())),
        preferred_element_type=jnp.float32)
    two_ab = 2.0 * jax.lax.dot_general(
        qk * qs, mk, (((0,), (0,)), ((), ())),
        preferred_element_type=jnp.float32)
    bsq = jnp.sum(qs * qk * qk, axis=0)[:, None]
    sim = (-a_sq + two_ab - bsq) * shr_ref[...] / math.sqrt(_CK)

    cm = jnp.max(sim.reshape(_QT, 128, 128), axis=2)

    def dbody(_, m):
        cand = jnp.where(cm < m, cm, -jnp.inf)
        return jnp.max(cand, axis=1, keepdims=True)

    th0 = jax.lax.fori_loop(
        0, _TOPK, dbody, jnp.full((_QT, 1), jnp.inf, jnp.float32))

    def cond(state):
        _, n = state
        return jnp.any(n > _TOPK)

    def wbody(state):
        th, n = state
        nxt = jnp.min(jnp.where(sim > th, sim, jnp.inf), axis=1,
                      keepdims=True)
        th2 = jnp.where(n > _TOPK, nxt, th)
        n2 = jnp.sum((sim >= th2).astype(jnp.float32), axis=1,
                     keepdims=True)
        return th2, n2

    n0 = jnp.sum((sim >= th0).astype(jnp.float32), axis=1, keepdims=True)
    thr, _ = jax.lax.while_loop(cond, wbody, (th0, n0))
    p = jnp.where(sim >= thr, jnp.exp(sim), 0.0)
    aff_ref[...] = (p / jnp.sum(p, axis=1, keepdims=True)).astype(
        jnp.bfloat16)


def _readout_kernel(aff_ref, vt_ref, out_ref):
    @pl.when(pl.program_id(0) == 0)
    def _():
        out_ref[...] = jnp.zeros_like(out_ref)

    out_ref[...] += jnp.dot(
        aff_ref[...], vt_ref[...],
        preferred_element_type=jnp.float32)


def kernel(q_key, q_selection, mem_key, mem_shrinkage, mem_value):
    qk = q_key.reshape(_CK, _HW)
    qs = q_selection.reshape(_CK, _HW)
    mk = mem_key.reshape(_CK, _T)
    shr = mem_shrinkage.reshape(1, _T)
    vt = mem_value.reshape(_CV2, _T).T.astype(jnp.bfloat16)

    aff = pl.pallas_call(
        _affinity_kernel,
        grid=(_NQT,),
        in_specs=[
            pl.BlockSpec((_CK, _QT), lambda i: (0, i)),
            pl.BlockSpec((_CK, _QT), lambda i: (0, i)),
            pl.BlockSpec((_CK, _T), lambda i: (0, 0)),
            pl.BlockSpec((1, _T), lambda i: (0, 0)),
        ],
        out_specs=pl.BlockSpec((_QT, _T), lambda i: (i, 0)),
        out_shape=jax.ShapeDtypeStruct((_HW, _T), jnp.bfloat16),
    )(qk, qs, mk, shr)

    out = pl.pallas_call(
        _readout_kernel,
        grid=(_NKC,),
        in_specs=[
            pl.BlockSpec((_HW, _KC), lambda k: (0, k)),
            pl.BlockSpec((_KC, _CV2), lambda k: (k, 0)),
        ],
        out_specs=pl.BlockSpec((_HW, _CV2), lambda k: (0, 0)),
        out_shape=jax.ShapeDtypeStruct((_HW, _CV2), jnp.float32),
    )(aff, vt)

    return out.T.reshape(2, 512, 32, 32)


# speedup vs baseline: 2.3667x; 2.3667x over previous
import math

import jax
import jax.numpy as jnp
from jax.experimental import pallas as pl

_CK = 64
_HW = 1024
_T = 16384
_TOPK = 30
_QT = 128
_NQT = _HW // _QT
_KC = 1024
_NKC = _T // _KC
_CV2 = 1024


def _affinity_kernel(qk_ref, qs_ref, mk_ref, shr_ref, aff_ref):
    qk = qk_ref[...]
    qs = qs_ref[...]
    mk = mk_ref[...]
    a_sq = jax.lax.dot_general(
        qs, mk * mk, (((0,), (0,)), ((), ())),
        preferred_element_type=jnp.float32)
    two_ab = 2.0 * jax.lax.dot_general(
        qk * qs, mk, (((0,), (0,)), ((), ())),
        preferred_element_type=jnp.float32)
    bsq = jnp.sum(qs * qk * qk, axis=0)[:, None]
    sim = (-a_sq + two_ab - bsq) * shr_ref[...] / math.sqrt(_CK)

    def body(_, m):
        cand = jnp.where(sim < m, sim, -jnp.inf)
        return jnp.max(cand, axis=1, keepdims=True)

    thr = jax.lax.fori_loop(
        0, _TOPK, body, jnp.full((_QT, 1), jnp.inf, jnp.float32))
    p = jnp.where(sim >= thr, jnp.exp(sim), 0.0)
    aff_ref[...] = (p / jnp.sum(p, axis=1, keepdims=True)).astype(
        jnp.bfloat16)


def _readout_kernel(aff_ref, vt_ref, out_ref):
    @pl.when(pl.program_id(0) == 0)
    def _():
        out_ref[...] = jnp.zeros_like(out_ref)

    out_ref[...] += jnp.dot(
        aff_ref[...], vt_ref[...],
        preferred_element_type=jnp.float32)


def kernel(q_key, q_selection, mem_key, mem_shrinkage, mem_value):
    qk = q_key.reshape(_CK, _HW)
    qs = q_selection.reshape(_CK, _HW)
    mk = mem_key.reshape(_CK, _T)
    shr = mem_shrinkage.reshape(1, _T)
    vt = mem_value.reshape(_CV2, _T).T.astype(jnp.bfloat16)

    aff = pl.pallas_call(
        _affinity_kernel,
        grid=(_NQT,),
        in_specs=[
            pl.BlockSpec((_CK, _QT), lambda i: (0, i)),
            pl.BlockSpec((_CK, _QT), lambda i: (0, i)),
            pl.BlockSpec((_CK, _T), lambda i: (0, 0)),
            pl.BlockSpec((1, _T), lambda i: (0, 0)),
        ],
        out_specs=pl.BlockSpec((_QT, _T), lambda i: (i, 0)),
        out_shape=jax.ShapeDtypeStruct((_HW, _T), jnp.bfloat16),
    )(qk, qs, mk, shr)

    out = pl.pallas_call(
        _readout_kernel,
        grid=(_NKC,),
        in_specs=[
            pl.BlockSpec((_HW, _KC), lambda k: (0, k)),
            pl.BlockSpec((_KC, _CV2), lambda k: (k, 0)),
        ],
        out_specs=pl.BlockSpec((_HW, _CV2), lambda k: (0, 0)),
        out_shape=jax.ShapeDtypeStruct((_HW, _CV2), jnp.float32),
    )(aff, vt)

    return out.T.reshape(2, 512, 32, 32)


# speedup vs baseline: 4.1603x; 1.7578x over previous
import math

import jax
import jax.numpy as jnp
from jax.experimental import pallas as pl

_CK = 64
_HW = 1024
_T = 16384
_TOPK = 30
_QT = 128
_NQT = _HW // _QT
_KC = 1024
_NKC = _T // _KC
_CV2 = 1024


def _affinity_kernel(qk_ref, qs_ref, mk_ref, shr_ref, aff_ref):
    qk = qk_ref[...]
    qs = qs_ref[...]
    mk = mk_ref[...]
    a_sq = jax.lax.dot_general(
        qs, mk * mk, (((0,), (0,)), ((), ())),
        preferred_element_type=jnp.float32)
    two_ab = 2.0 * jax.lax.dot_general(
        qk * qs, mk, (((0,), (0,)), ((), ())),
        preferred_element_type=jnp.float32)
    bsq = jnp.sum(qs * qk * qk, axis=0)[:, None]
    sim = (-a_sq + two_ab - bsq) * shr_ref[...] / math.sqrt(_CK)

    cm = jnp.maximum(
        jnp.maximum(jnp.maximum(sim[:, 0:2048], sim[:, 2048:4096]),
                    jnp.maximum(sim[:, 4096:6144], sim[:, 6144:8192])),
        jnp.maximum(jnp.maximum(sim[:, 8192:10240], sim[:, 10240:12288]),
                    jnp.maximum(sim[:, 12288:14336], sim[:, 14336:16384])))

    def dbody(_, m):
        cand = jnp.where(cm < m, cm, -jnp.inf)
        return jnp.max(cand, axis=1, keepdims=True)

    th0 = jax.lax.fori_loop(
        0, _TOPK, dbody, jnp.full((_QT, 1), jnp.inf, jnp.float32))

    def cond(state):
        _, n = state
        return jnp.any(n > _TOPK)

    def wbody(state):
        th, n = state
        nxt = jnp.min(jnp.where(sim > th, sim, jnp.inf), axis=1,
                      keepdims=True)
        th2 = jnp.where(n > _TOPK, nxt, th)
        n2 = jnp.sum((sim >= th2).astype(jnp.float32), axis=1,
                     keepdims=True)
        return th2, n2

    n0 = jnp.sum((sim >= th0).astype(jnp.float32), axis=1, keepdims=True)
    thr, _ = jax.lax.while_loop(cond, wbody, (th0, n0))
    p = jnp.where(sim >= thr, jnp.exp(sim), 0.0)
    aff_ref[...] = p / jnp.sum(p, axis=1, keepdims=True)


def _readout_kernel(aff_ref, vt_ref, out_ref):
    @pl.when(pl.program_id(0) == 0)
    def _():
        out_ref[...] = jnp.zeros_like(out_ref)

    out_ref[...] += jnp.dot(
        aff_ref[...], vt_ref[...],
        preferred_element_type=jnp.float32)


def kernel(q_key, q_selection, mem_key, mem_shrinkage, mem_value):
    qk = q_key.reshape(_CK, _HW)
    qs = q_selection.reshape(_CK, _HW)
    mk = mem_key.reshape(_CK, _T)
    shr = mem_shrinkage.reshape(1, _T)
    vt = mem_value.reshape(_CV2, _T).T

    aff = pl.pallas_call(
        _affinity_kernel,
        grid=(_NQT,),
        in_specs=[
            pl.BlockSpec((_CK, _QT), lambda i: (0, i)),
            pl.BlockSpec((_CK, _QT), lambda i: (0, i)),
            pl.BlockSpec((_CK, _T), lambda i: (0, 0)),
            pl.BlockSpec((1, _T), lambda i: (0, 0)),
        ],
        out_specs=pl.BlockSpec((_QT, _T), lambda i: (i, 0)),
        out_shape=jax.ShapeDtypeStruct((_HW, _T), jnp.float32),
    )(qk, qs, mk, shr)

    out = pl.pallas_call(
        _readout_kernel,
        grid=(_NKC,),
        in_specs=[
            pl.BlockSpec((_HW, _KC), lambda k: (0, k)),
            pl.BlockSpec((_KC, _CV2), lambda k: (k, 0)),
        ],
        out_specs=pl.BlockSpec((_HW, _CV2), lambda k: (0, 0)),
        out_shape=jax.ShapeDtypeStruct((_HW, _CV2), jnp.float32),
    )(aff, vt)

    return out.T.reshape(2, 512, 32, 32)
